# baseline (speedup 1.0000x reference)
import functools

import jax
import jax.numpy as jnp
from jax import lax
from jax.experimental import pallas as pl
from jax.experimental.pallas import tpu as pltpu
from jax.experimental.pallas import tpu_sc as plsc

N = 10000
E = 320000
G = 64
Dm = 128
H = 128
C = 2
EPS = 1e-5

NC = 2
NS = 16
NW = NC * NS
LANES = 16

CH = 128
NCHUNKS = 80
E_PAD = NW * NCHUNKS * CH
NPAD = 10112
RPT = NPAD // NS

BR = 1000
NBLK = N // BR

_MESH = dict(core_axis_name="c", subcore_axis_name="s")


def _zero_fill(buf, rows, width):
    def fill(t, _):
        i = t // (width // LANES)
        k = t % (width // LANES)
        buf[i, pl.ds(k * LANES, LANES)] = jnp.zeros((LANES,), jnp.float32)
        return 0
    lax.fori_loop(0, rows * (width // LANES), fill, 0)


def _zero_stripe(buf, dst_at, rows):
    nfull = rows // CH
    rem = rows - nfull * CH

    def step(j, _):
        pltpu.sync_copy(buf, dst_at(j * CH, CH))
        return 0
    lax.fori_loop(0, nfull, step, 0)
    if rem:
        pltpu.sync_copy(buf.at[pl.ds(0, rem)], dst_at(nfull * CH, rem))


def _stripe_copy(src_at, dst_at, buf, rows):
    nfull = rows // CH
    rem = rows - nfull * CH

    def step(j, _):
        pltpu.sync_copy(src_at(j * CH, CH), buf)
        pltpu.sync_copy(buf, dst_at(j * CH, CH))
        return 0
    lax.fori_loop(0, nfull, step, 0)
    if rem:
        pltpu.sync_copy(src_at(nfull * CH, rem), buf.at[pl.ds(0, rem)])
        pltpu.sync_copy(buf.at[pl.ds(0, rem)], dst_at(nfull * CH, rem))


def _sc_degree(dst3):
    @functools.partial(
        pl.kernel,
        out_type=jax.ShapeDtypeStruct((NC, NPAD, LANES), jnp.float32),
        mesh=plsc.VectorSubcoreMesh(**_MESH),
        scratch_types=[
            pltpu.VMEM((NCHUNKS, CH), jnp.int32),
            pltpu.VMEM((CH, LANES), jnp.float32),
            pltpu.VMEM((CH, LANES), jnp.float32),
            pltpu.VMEM_SHARED((NPAD, LANES), jnp.float32),
        ],
    )
    def k(dst_hbm, out_hbm, idx_v, ones_v, buf_v, acc_sh):
        cid = lax.axis_index("c")
        sid = lax.axis_index("s")
        wid = cid * NS + sid

        def fill(i, _):
            ones_v[i, :] = jnp.ones((LANES,), jnp.float32)
            return 0
        lax.fori_loop(0, CH, fill, 0)
        _zero_fill(buf_v, CH, LANES)

        base = sid * RPT
        _zero_stripe(buf_v, lambda o, n: acc_sh.at[pl.ds(base + o, n)], RPT)
        plsc.subcore_barrier()

        pltpu.sync_copy(dst_hbm.at[wid], idx_v)

        def body(j, _):
            pltpu.sync_copy(ones_v, acc_sh.at[idx_v.at[j]], add=True)
            return 0
        lax.fori_loop(0, NCHUNKS, body, 0)
        plsc.subcore_barrier()

        _stripe_copy(lambda o, n: acc_sh.at[pl.ds(base + o, n)],
                     lambda o, n: out_hbm.at[cid].at[pl.ds(base + o, n)],
                     buf_v, RPT)

    return k(dst3)


def _sc_propagate(y, src3, dst3):
    @functools.partial(
        pl.kernel,
        out_type=jax.ShapeDtypeStruct((NC, NPAD, Dm), jnp.float32),
        mesh=plsc.VectorSubcoreMesh(**_MESH),
        scratch_types=[
            pltpu.VMEM((NCHUNKS // 2, CH), jnp.int32),
            pltpu.VMEM((NCHUNKS // 2, CH), jnp.int32),
            pltpu.VMEM((CH, Dm), jnp.float32),
            pltpu.VMEM((CH, Dm), jnp.float32),
            pltpu.SemaphoreType.DMA,
            pltpu.SemaphoreType.DMA,
            pltpu.VMEM_SHARED((NPAD, Dm), jnp.float32),
        ],
    )
    def k(y_hbm, src_hbm, dst_hbm, out_hbm, si_v, di_v, rows0, rows1,
          sem0, sem1, acc_sh):
        cid = lax.axis_index("c")
        sid = lax.axis_index("s")
        wid = cid * NS + sid
        bufs = ((rows0, sem0), (rows1, sem1))
        half = NCHUNKS // 2

        _zero_fill(rows0, CH, Dm)
        base = sid * RPT
        _zero_stripe(rows0, lambda o, n: acc_sh.at[pl.ds(base + o, n)], RPT)
        plsc.subcore_barrier()

        for h in range(2):
            pltpu.sync_copy(src_hbm.at[wid].at[pl.ds(h * half, half)], si_v)
            pltpu.sync_copy(dst_hbm.at[wid].at[pl.ds(h * half, half)], di_v)
            pltpu.async_copy(y_hbm.at[si_v.at[0]], rows0, sem0)
            pltpu.async_copy(y_hbm.at[si_v.at[1]], rows1, sem1)

            def body(i2, _):
                for b, (buf, sem) in enumerate(bufs):
                    j = 2 * i2 + b
                    pltpu.make_async_copy(
                        y_hbm.at[si_v.at[j]], buf, sem).wait()
                    pltpu.sync_copy(buf, acc_sh.at[di_v.at[j]], add=True)
                    pltpu.async_copy(y_hbm.at[si_v.at[j + 2]], buf, sem)
                return 0
            lax.fori_loop(0, half // 2 - 1, body, 0)
            for b, (buf, sem) in enumerate(bufs):
                j = half - 2 + b
                pltpu.make_async_copy(y_hbm.at[si_v.at[j]], buf, sem).wait()
                pltpu.sync_copy(buf, acc_sh.at[di_v.at[j]], add=True)
        plsc.subcore_barrier()

        _stripe_copy(lambda o, n: acc_sh.at[pl.ds(base + o, n)],
                     lambda o, n: out_hbm.at[cid].at[pl.ds(base + o, n)],
                     rows0, RPT)

    return k(y, src3, dst3)


def _tc_pre(x, W1, degA, degB):
    def body(x_ref, w_ref, da_ref, db_ref, p_ref, y_ref):
        d = da_ref[:, 0:1] + db_ref[:, 0:1] + 1.0
        dinv = lax.rsqrt(d)
        p = jnp.dot(x_ref[...], w_ref[...], preferred_element_type=jnp.float32)
        p_ref[...] = p
        y_ref[...] = p * dinv

    return pl.pallas_call(
        body,
        grid=(NBLK,),
        in_specs=[
            pl.BlockSpec((BR, Dm), lambda i: (i, 0)),
            pl.BlockSpec((Dm, H), lambda i: (0, 0)),
            pl.BlockSpec((BR, LANES), lambda i: (i, 0)),
            pl.BlockSpec((BR, LANES), lambda i: (i, 0)),
        ],
        out_specs=[pl.BlockSpec((BR, H), lambda i: (i, 0))] * 2,
        out_shape=[jax.ShapeDtypeStruct((N, H), jnp.float32)] * 2,
    )(x, W1, degA, degB)


def _tc_mid(Sa, Sb, p, degA, degB, b, g, bt, m, v, Wn):
    def body(sa, sb, pr, da, db, b_r, g_r, bt_r, m_r, v_r, w_r,
             pn_ref, yn_ref):
        d = da[:, 0:1] + db[:, 0:1] + 1.0
        dinv = lax.rsqrt(d)
        z = dinv * (sa[...] + sb[...]) + (dinv * dinv) * pr[...] + b_r[...]
        z = (z - m_r[...]) * (g_r[...] * lax.rsqrt(v_r[...] + EPS)) + bt_r[...]
        h = jnp.maximum(z, 0.0)
        pn = jnp.dot(h, w_r[...], preferred_element_type=jnp.float32)
        pn_ref[...] = pn
        yn_ref[...] = pn * dinv

    vec = pl.BlockSpec((1, H), lambda i: (0, 0))
    blk = pl.BlockSpec((BR, H), lambda i: (i, 0))
    deg = pl.BlockSpec((BR, LANES), lambda i: (i, 0))
    return pl.pallas_call(
        body,
        grid=(NBLK,),
        in_specs=[blk, blk, blk, deg, deg, vec, vec, vec, vec, vec,
                  pl.BlockSpec((H, H), lambda i: (0, 0))],
        out_specs=[blk, blk],
        out_shape=[jax.ShapeDtypeStruct((N, H), jnp.float32)] * 2,
    )(Sa, Sb, p, degA, degB, b, g, bt, m, v, Wn)


def _tc_final(Sa, Sb, p, degA, degB, b, g, bt, m, v, batch2,
              Wc1p, bc1p, Wc2p, bc2p):
    def body(sa, sb, pr, da, db, b_r, g_r, bt_r, m_r, v_r, bat_r,
             w1_r, b1_r, w2_r, b2_r, out_ref, seg_acc, cnt_acc):
        i = pl.program_id(0)
        d = da[:, 0:1] + db[:, 0:1] + 1.0
        dinv = lax.rsqrt(d)
        z = dinv * (sa[...] + sb[...]) + (dinv * dinv) * pr[...] + b_r[...]
        h = (z - m_r[...]) * (g_r[...] * lax.rsqrt(v_r[...] + EPS)) + bt_r[...]

        gid = lax.broadcasted_iota(jnp.int32, (G, BR), 0)
        onehot = (bat_r[0] == gid).astype(jnp.float32)

        @pl.when(i == 0)
        def _():
            seg_acc[...] = jnp.zeros_like(seg_acc)
            cnt_acc[...] = jnp.zeros_like(cnt_acc)

        seg_acc[...] += jnp.dot(onehot, h, preferred_element_type=jnp.float32)
        cnt_acc[...] += jnp.broadcast_to(
            jnp.sum(onehot, axis=1, keepdims=True), (G, H))

        @pl.when(i == NBLK - 1)
        def _():
            emb = seg_acc[...] / jnp.maximum(cnt_acc[...], 1.0)
            zz = jnp.maximum(
                jnp.dot(emb, w1_r[...], preferred_element_type=jnp.float32)
                + b1_r[...], 0.0)
            out_ref[...] = (
                jnp.dot(zz, w2_r[...], preferred_element_type=jnp.float32)
                + b2_r[...])

    vec = pl.BlockSpec((1, H), lambda i: (0, 0))
    blk = pl.BlockSpec((BR, H), lambda i: (i, 0))
    deg = pl.BlockSpec((BR, LANES), lambda i: (i, 0))
    return pl.pallas_call(
        body,
        grid=(NBLK,),
        in_specs=[blk, blk, blk, deg, deg, vec, vec, vec, vec, vec,
                  pl.BlockSpec((1, 1, BR), lambda i: (i, 0, 0)),
                  pl.BlockSpec((H, H), lambda i: (0, 0)), vec,
                  pl.BlockSpec((H, H), lambda i: (0, 0)), vec],
        out_specs=pl.BlockSpec((G, H), lambda i: (0, 0)),
        out_shape=jax.ShapeDtypeStruct((G, H), jnp.float32),
        scratch_shapes=[pltpu.VMEM((G, H), jnp.float32),
                        pltpu.VMEM((G, H), jnp.float32)],
    )(Sa, Sb, p, degA, degB, b, g, bt, m, v, batch2, Wc1p, bc1p, Wc2p, bc2p)


def kernel(x, edge_index, batch, W1, b1, W2, b2, W3, b3,
           g1, bt1, m1, v1, g2, bt2, m2, v2, g3, bt3, m3, v3,
           Wc1, bc1, Wc2, bc2):
    src = edge_index[0].astype(jnp.int32)
    dst = edge_index[1].astype(jnp.int32)
    ppt = NCHUNKS * CH - E // NW
    src3 = jnp.pad(src.reshape(NW, E // NW),
                   ((0, 0), (0, ppt))).reshape(NW, NCHUNKS, CH)
    dst3 = jnp.pad(dst.reshape(NW, E // NW), ((0, 0), (0, ppt)),
                   constant_values=N).reshape(NW, NCHUNKS, CH)

    degP = _sc_degree(dst3)
    degA = degP[0, :N, :]
    degB = degP[1, :N, :]

    b1r, b2r, b3r = (t.reshape(1, H) for t in (b1, b2, b3))
    g1r, g2r, g3r = (t.reshape(1, H) for t in (g1, g2, g3))
    t1r, t2r, t3r = (t.reshape(1, H) for t in (bt1, bt2, bt3))
    m1r, m2r, m3r = (t.reshape(1, H) for t in (m1, m2, m3))
    v1r, v2r, v3r = (t.reshape(1, H) for t in (v1, v2, v3))

    p1, y1 = _tc_pre(x, W1, degA, degB)
    S1 = _sc_propagate(y1, src3, dst3)
    p2, y2 = _tc_mid(S1[0, :N], S1[1, :N], p1, degA, degB,
                     b1r, g1r, t1r, m1r, v1r, W2)
    S2 = _sc_propagate(y2, src3, dst3)
    p3, y3 = _tc_mid(S2[0, :N], S2[1, :N], p2, degA, degB,
                     b2r, g2r, t2r, m2r, v2r, W3)
    S3 = _sc_propagate(y3, src3, dst3)

    batch2 = batch.astype(jnp.int32).reshape(NBLK, 1, BR)
    Wc1p = jnp.pad(Wc1, ((0, 0), (0, H - Wc1.shape[1])))
    bc1p = jnp.pad(bc1, (0, H - bc1.shape[0])).reshape(1, H)
    Wc2p = jnp.pad(Wc2, ((0, H - Wc2.shape[0]), (0, H - Wc2.shape[1])))
    bc2p = jnp.pad(bc2, (0, H - bc2.shape[0])).reshape(1, H)

    logits_pad = _tc_final(S3[0, :N], S3[1, :N], p3, degA, degB,
                           b3r, g3r, t3r, m3r, v3r, batch2,
                           Wc1p, bc1p, Wc2p, bc2p)
    return logits_pad[:, :C]


# speedup vs baseline: 2.8132x; 2.8132x over previous
import functools

import jax
import jax.numpy as jnp
from jax import lax
from jax.experimental import pallas as pl
from jax.experimental.pallas import tpu as pltpu
from jax.experimental.pallas import tpu_sc as plsc

N = 10000
E = 320000
G = 64
Dm = 128
H = 128
C = 2
EPS = 1e-5

NC = 2
NS = 16
NW = NC * NS
LANES = 16

CH = 128
CHD = 125
NCHUNKS = 80
SPC = 120
NPAD = 10112
RPT = NPAD // NS

BR = 1000
NBLK = N // BR

_MESH = dict(core_axis_name="c", subcore_axis_name="s")


def _zero_fill(buf, rows, width):
    def fill(t, _):
        i = t // (width // LANES)
        k = t % (width // LANES)
        buf[i, pl.ds(k * LANES, LANES)] = jnp.zeros((LANES,), jnp.float32)
        return 0
    lax.fori_loop(0, rows * (width // LANES), fill, 0)


def _zero_stripe(buf, dst_at, rows):
    nfull = rows // SPC
    rem = rows - nfull * SPC

    def step(j, _):
        pltpu.sync_copy(buf.at[pl.ds(0, SPC)], dst_at(j * SPC, SPC))
        return 0
    lax.fori_loop(0, nfull, step, 0)
    if rem:
        pltpu.sync_copy(buf.at[pl.ds(0, rem)], dst_at(nfull * SPC, rem))


def _stripe_copy(src_at, dst_at, buf, rows):
    nfull = rows // SPC
    rem = rows - nfull * SPC

    def step(j, _):
        pltpu.sync_copy(src_at(j * SPC, SPC), buf.at[pl.ds(0, SPC)])
        pltpu.sync_copy(buf.at[pl.ds(0, SPC)], dst_at(j * SPC, SPC))
        return 0
    lax.fori_loop(0, nfull, step, 0)
    if rem:
        pltpu.sync_copy(src_at(nfull * SPC, rem), buf.at[pl.ds(0, rem)])
        pltpu.sync_copy(buf.at[pl.ds(0, rem)], dst_at(nfull * SPC, rem))


def _sc_degree(dst3):
    @functools.partial(
        pl.kernel,
        out_type=jax.ShapeDtypeStruct((NC, NPAD, LANES), jnp.float32),
        mesh=plsc.VectorSubcoreMesh(**_MESH),
        scratch_types=[
            pltpu.VMEM((NCHUNKS, CH), jnp.int32),
            pltpu.VMEM((CHD, LANES), jnp.float32),
            pltpu.VMEM((CHD, LANES), jnp.float32),
            pltpu.VMEM_SHARED((NPAD, LANES), jnp.float32),
        ],
    )
    def k(dst_hbm, out_hbm, idx_v, ones_v, buf_v, acc_sh):
        cid = lax.axis_index("c")
        sid = lax.axis_index("s")
        wid = cid * NS + sid

        def fill(i, _):
            ones_v[i, :] = jnp.ones((LANES,), jnp.float32)
            return 0
        lax.fori_loop(0, CHD, fill, 0)
        _zero_fill(buf_v, CHD, LANES)

        base = sid * RPT
        _zero_stripe(buf_v, lambda o, n: acc_sh.at[pl.ds(base + o, n)], RPT)
        plsc.subcore_barrier()

        pltpu.sync_copy(dst_hbm.at[wid], idx_v)

        def body(j, _):
            pltpu.sync_copy(ones_v,
                            acc_sh.at[idx_v.at[j].at[pl.ds(0, CHD)]],
                            add=True)
            return 0
        lax.fori_loop(0, NCHUNKS, body, 0)
        plsc.subcore_barrier()

        _stripe_copy(lambda o, n: acc_sh.at[pl.ds(base + o, n)],
                     lambda o, n: out_hbm.at[cid].at[pl.ds(base + o, n)],
                     buf_v, RPT)

    return k(dst3)


def _sc_propagate(y, src3, dst3):
    @functools.partial(
        pl.kernel,
        out_type=jax.ShapeDtypeStruct((NC, NPAD, Dm), jnp.float32),
        mesh=plsc.VectorSubcoreMesh(**_MESH),
        scratch_types=[
            pltpu.VMEM((NCHUNKS // 2, CH), jnp.int32),
            pltpu.VMEM((NCHUNKS // 2, CH), jnp.int32),
            pltpu.VMEM((CHD, Dm), jnp.float32),
            pltpu.VMEM((CHD, Dm), jnp.float32),
            pltpu.SemaphoreType.DMA,
            pltpu.SemaphoreType.DMA,
            pltpu.VMEM_SHARED((NPAD, Dm), jnp.float32),
        ],
    )
    def k(y_hbm, src_hbm, dst_hbm, out_hbm, si_v, di_v, rows0, rows1,
          sem0, sem1, acc_sh):
        cid = lax.axis_index("c")
        sid = lax.axis_index("s")
        wid = cid * NS + sid
        bufs = ((rows0, sem0), (rows1, sem1))
        half = NCHUNKS // 2

        _zero_fill(rows0, CHD, Dm)
        base = sid * RPT
        _zero_stripe(rows0, lambda o, n: acc_sh.at[pl.ds(base + o, n)], RPT)
        plsc.subcore_barrier()

        for h in range(2):
            pltpu.sync_copy(src_hbm.at[wid].at[pl.ds(h * half, half)], si_v)
            pltpu.sync_copy(dst_hbm.at[wid].at[pl.ds(h * half, half)], di_v)
            pltpu.async_copy(
                y_hbm.at[si_v.at[0].at[pl.ds(0, CHD)]], rows0, sem0)
            pltpu.async_copy(
                y_hbm.at[si_v.at[1].at[pl.ds(0, CHD)]], rows1, sem1)

            def body(i2, _):
                for b, (buf, sem) in enumerate(bufs):
                    j = 2 * i2 + b
                    pltpu.make_async_copy(
                        y_hbm.at[si_v.at[j].at[pl.ds(0, CHD)]],
                        buf, sem).wait()
                    pltpu.sync_copy(
                        buf, acc_sh.at[di_v.at[j].at[pl.ds(0, CHD)]],
                        add=True)
                    pltpu.async_copy(
                        y_hbm.at[si_v.at[j + 2].at[pl.ds(0, CHD)]], buf, sem)
                return 0
            lax.fori_loop(0, half // 2 - 1, body, 0)
            for b, (buf, sem) in enumerate(bufs):
                j = half - 2 + b
                pltpu.make_async_copy(
                    y_hbm.at[si_v.at[j].at[pl.ds(0, CHD)]], buf, sem).wait()
                pltpu.sync_copy(
                    buf, acc_sh.at[di_v.at[j].at[pl.ds(0, CHD)]], add=True)
        plsc.subcore_barrier()

        _stripe_copy(lambda o, n: acc_sh.at[pl.ds(base + o, n)],
                     lambda o, n: out_hbm.at[cid].at[pl.ds(base + o, n)],
                     rows0, RPT)

    return k(y, src3, dst3)


def _tc_pre(x, W1, degA, degB):
    def body(x_ref, w_ref, da_ref, db_ref, p_ref, y_ref):
        d = da_ref[:, 0:1] + db_ref[:, 0:1] + 1.0
        dinv = lax.rsqrt(d)
        p = jnp.dot(x_ref[...], w_ref[...], preferred_element_type=jnp.float32)
        p_ref[...] = p
        y_ref[...] = p * dinv

    return pl.pallas_call(
        body,
        grid=(NBLK,),
        in_specs=[
            pl.BlockSpec((BR, Dm), lambda i: (i, 0)),
            pl.BlockSpec((Dm, H), lambda i: (0, 0)),
            pl.BlockSpec((BR, LANES), lambda i: (i, 0)),
            pl.BlockSpec((BR, LANES), lambda i: (i, 0)),
        ],
        out_specs=[pl.BlockSpec((BR, H), lambda i: (i, 0))] * 2,
        out_shape=[jax.ShapeDtypeStruct((N, H), jnp.float32)] * 2,
    )(x, W1, degA, degB)


def _tc_mid(Sa, Sb, p, degA, degB, b, g, bt, m, v, Wn):
    def body(sa, sb, pr, da, db, b_r, g_r, bt_r, m_r, v_r, w_r,
             pn_ref, yn_ref):
        d = da[:, 0:1] + db[:, 0:1] + 1.0
        dinv = lax.rsqrt(d)
        z = dinv * (sa[...] + sb[...]) + (dinv * dinv) * pr[...] + b_r[...]
        z = (z - m_r[...]) * (g_r[...] * lax.rsqrt(v_r[...] + EPS)) + bt_r[...]
        h = jnp.maximum(z, 0.0)
        pn = jnp.dot(h, w_r[...], preferred_element_type=jnp.float32)
        pn_ref[...] = pn
        yn_ref[...] = pn * dinv

    vec = pl.BlockSpec((1, H), lambda i: (0, 0))
    blk = pl.BlockSpec((BR, H), lambda i: (i, 0))
    deg = pl.BlockSpec((BR, LANES), lambda i: (i, 0))
    return pl.pallas_call(
        body,
        grid=(NBLK,),
        in_specs=[blk, blk, blk, deg, deg, vec, vec, vec, vec, vec,
                  pl.BlockSpec((H, H), lambda i: (0, 0))],
        out_specs=[blk, blk],
        out_shape=[jax.ShapeDtypeStruct((N, H), jnp.float32)] * 2,
    )(Sa, Sb, p, degA, degB, b, g, bt, m, v, Wn)


def _tc_final(Sa, Sb, p, degA, degB, b, g, bt, m, v, batch2,
              Wc1p, bc1p, Wc2p, bc2p):
    def body(sa, sb, pr, da, db, b_r, g_r, bt_r, m_r, v_r, bat_r,
             w1_r, b1_r, w2_r, b2_r, out_ref, seg_acc, cnt_acc):
        i = pl.program_id(0)
        d = da[:, 0:1] + db[:, 0:1] + 1.0
        dinv = lax.rsqrt(d)
        z = dinv * (sa[...] + sb[...]) + (dinv * dinv) * pr[...] + b_r[...]
        h = (z - m_r[...]) * (g_r[...] * lax.rsqrt(v_r[...] + EPS)) + bt_r[...]

        gid = lax.broadcasted_iota(jnp.int32, (G, BR), 0)
        onehot = (bat_r[0] == gid).astype(jnp.float32)

        @pl.when(i == 0)
        def _():
            seg_acc[...] = jnp.zeros_like(seg_acc)
            cnt_acc[...] = jnp.zeros_like(cnt_acc)

        seg_acc[...] += jnp.dot(onehot, h, preferred_element_type=jnp.float32)
        cnt_acc[...] += jnp.broadcast_to(
            jnp.sum(onehot, axis=1, keepdims=True), (G, H))

        @pl.when(i == NBLK - 1)
        def _():
            emb = seg_acc[...] / jnp.maximum(cnt_acc[...], 1.0)
            zz = jnp.maximum(
                jnp.dot(emb, w1_r[...], preferred_element_type=jnp.float32)
                + b1_r[...], 0.0)
            out_ref[...] = (
                jnp.dot(zz, w2_r[...], preferred_element_type=jnp.float32)
                + b2_r[...])

    vec = pl.BlockSpec((1, H), lambda i: (0, 0))
    blk = pl.BlockSpec((BR, H), lambda i: (i, 0))
    deg = pl.BlockSpec((BR, LANES), lambda i: (i, 0))
    return pl.pallas_call(
        body,
        grid=(NBLK,),
        in_specs=[blk, blk, blk, deg, deg, vec, vec, vec, vec, vec,
                  pl.BlockSpec((1, 1, BR), lambda i: (i, 0, 0)),
                  pl.BlockSpec((H, H), lambda i: (0, 0)), vec,
                  pl.BlockSpec((H, H), lambda i: (0, 0)), vec],
        out_specs=pl.BlockSpec((G, H), lambda i: (0, 0)),
        out_shape=jax.ShapeDtypeStruct((G, H), jnp.float32),
        scratch_shapes=[pltpu.VMEM((G, H), jnp.float32),
                        pltpu.VMEM((G, H), jnp.float32)],
    )(Sa, Sb, p, degA, degB, b, g, bt, m, v, batch2, Wc1p, bc1p, Wc2p, bc2p)


def kernel(x, edge_index, batch, W1, b1, W2, b2, W3, b3,
           g1, bt1, m1, v1, g2, bt2, m2, v2, g3, bt3, m3, v3,
           Wc1, bc1, Wc2, bc2):
    src = edge_index[0].astype(jnp.int32)
    dst = edge_index[1].astype(jnp.int32)
    src3 = jnp.pad(src.reshape(NW, NCHUNKS, CHD), ((0, 0), (0, 0), (0, CH - CHD)))
    dst3 = jnp.pad(dst.reshape(NW, NCHUNKS, CHD), ((0, 0), (0, 0), (0, CH - CHD)),
                   constant_values=N)

    degP = _sc_degree(dst3)
    degA = degP[0, :N, :]
    degB = degP[1, :N, :]

    b1r, b2r, b3r = (t.reshape(1, H) for t in (b1, b2, b3))
    g1r, g2r, g3r = (t.reshape(1, H) for t in (g1, g2, g3))
    t1r, t2r, t3r = (t.reshape(1, H) for t in (bt1, bt2, bt3))
    m1r, m2r, m3r = (t.reshape(1, H) for t in (m1, m2, m3))
    v1r, v2r, v3r = (t.reshape(1, H) for t in (v1, v2, v3))

    p1, y1 = _tc_pre(x, W1, degA, degB)
    S1 = _sc_propagate(y1, src3, dst3)
    p2, y2 = _tc_mid(S1[0, :N], S1[1, :N], p1, degA, degB,
                     b1r, g1r, t1r, m1r, v1r, W2)
    S2 = _sc_propagate(y2, src3, dst3)
    p3, y3 = _tc_mid(S2[0, :N], S2[1, :N], p2, degA, degB,
                     b2r, g2r, t2r, m2r, v2r, W3)
    S3 = _sc_propagate(y3, src3, dst3)

    batch2 = batch.astype(jnp.int32).reshape(NBLK, 1, BR)
    Wc1p = jnp.pad(Wc1, ((0, 0), (0, H - Wc1.shape[1])))
    bc1p = jnp.pad(bc1, (0, H - bc1.shape[0])).reshape(1, H)
    Wc2p = jnp.pad(Wc2, ((0, H - Wc2.shape[0]), (0, H - Wc2.shape[1])))
    bc2p = jnp.pad(bc2, (0, H - bc2.shape[0])).reshape(1, H)

    logits_pad = _tc_final(S3[0, :N], S3[1, :N], p3, degA, degB,
                           b3r, g3r, t3r, m3r, v3r, batch2,
                           Wc1p, bc1p, Wc2p, bc2p)
    return logits_pad[:, :C]
